# baseline (speedup 1.0000x reference)
import functools

import jax
import jax.numpy as jnp
from jax import lax
from jax.experimental import pallas as pl
from jax.experimental.pallas import tpu as pltpu
from jax.experimental.pallas import tpu_sc as plsc

_K = 1024
_D = 64
_N = 16384
_COMMIT = 0.25
_BN = 1024

_NC = 2
_NS = 16
_NW = _NC * _NS
_BW = _N // _NW
_CH = 128
_NCH = _BW // _CH


def _argmin_kernel(z_ref, w_ref, idx_ref):
    z = z_ref[...]
    w = w_ref[...]
    wm2 = w * (-2.0)
    s2 = jax.lax.dot_general(
        z, wm2, (((1,), (1,)), ((), ())), preferred_element_type=jnp.float32)
    zsq = jnp.sum(z * z, axis=1, keepdims=True)
    wsq = jnp.sum(w * w, axis=1)
    d = (zsq + wsq[None, :]) + s2
    idx = jnp.argmin(d, axis=1).astype(jnp.int32)
    idx_ref[...] = idx.reshape(_BN, 1)


def _sc_kernel(wp_hbm, idx_hbm, z2_hbm,
               q2_hbm, sq_hbm,
               idx_c0, idx_c1, idx_c2, idx_c3,
               rowsp_v, z_v, q_v, acc_v, sem):
    cid = lax.axis_index("c")
    sid = lax.axis_index("s")
    wid = sid * _NC + cid
    idx_chunks = [idx_c0, idx_c1, idx_c2, idx_c3]

    for j in range(_NCH):
        pltpu.sync_copy(idx_hbm.at[wid * _NCH + j], idx_chunks[j])

    acc = jnp.zeros((16,), jnp.float32)
    for j in range(_NCH):
        pltpu.async_copy(wp_hbm.at[idx_chunks[j]], rowsp_v, sem).wait()
        zbase = wid * (_BW // 2) + j * (_CH // 2)
        pltpu.sync_copy(z2_hbm.at[pl.ds(zbase, _CH // 2)], z_v)

        def body(u, acc):
            for half in range(2):
                t = 2 * u + half
                for c in range(_D // 16):
                    r = rowsp_v[t, pl.ds(c * 16, 16)]
                    zz = z_v[u, pl.ds(half * _D + c * 16, 16)]
                    v = r - zz
                    acc = acc + v * v
                    q_v[u, pl.ds(half * _D + c * 16, 16)] = r
            return acc

        acc = lax.fori_loop(0, _CH // 2, body, acc)
        pltpu.sync_copy(q_v, q2_hbm.at[pl.ds(zbase, _CH // 2)])

    acc_v[...] = acc
    pltpu.sync_copy(acc_v, sq_hbm.at[wid])


def _stats_kernel(idx_ref, sq_ref, loss_ref, ppl_ref, counts_ref):
    i = pl.program_id(0)
    nblk = pl.num_programs(0)
    idx = idx_ref[...]
    col = jax.lax.broadcasted_iota(jnp.int32, (_BN, _K), 1)
    onehot = (col == idx).astype(jnp.float32)
    ones_row = jnp.ones((1, _BN), jnp.float32)
    blk_counts = jax.lax.dot_general(
        ones_row, onehot, (((1,), (0,)), ((), ())),
        preferred_element_type=jnp.float32)

    @pl.when(i == 0)
    def _init():
        counts_ref[...] = blk_counts

    @pl.when(i > 0)
    def _acc():
        counts_ref[...] += blk_counts

    @pl.when(i == nblk - 1)
    def _final():
        p = counts_ref[...] / jnp.float32(_N)
        ent = -jnp.sum(p * jnp.log(p + 1e-10))
        ppl_ref[...] = jnp.full((1, 1), jnp.exp(ent), jnp.float32)
        sq = jnp.sum(sq_ref[...])
        loss_ref[...] = jnp.full(
            (1, 1), (1.0 + _COMMIT) * sq / jnp.float32(_N * _D), jnp.float32)


def _make_sc_call():
    mesh = plsc.VectorSubcoreMesh(core_axis_name="c", subcore_axis_name="s")
    return functools.partial(
        pl.kernel,
        mesh=mesh,
        out_type=[
            jax.ShapeDtypeStruct((_N // 2, 2 * _D), jnp.float32),
            jax.ShapeDtypeStruct((_NW, 16), jnp.float32),
        ],
        scratch_types=[
            pltpu.VMEM((_CH,), jnp.int32),
            pltpu.VMEM((_CH,), jnp.int32),
            pltpu.VMEM((_CH,), jnp.int32),
            pltpu.VMEM((_CH,), jnp.int32),
            pltpu.VMEM((_CH, 2 * _D), jnp.float32),
            pltpu.VMEM((_CH // 2, 2 * _D), jnp.float32),
            pltpu.VMEM((_CH // 2, 2 * _D), jnp.float32),
            pltpu.VMEM((16,), jnp.float32),
            pltpu.SemaphoreType.DMA,
        ],
    )(_sc_kernel)


def kernel(inputs, W):
    n = inputs.shape[0]
    grid = (n // _BN,)
    idx2d = pl.pallas_call(
        _argmin_kernel,
        grid=grid,
        in_specs=[
            pl.BlockSpec((_BN, _D), lambda i: (i, 0)),
            pl.BlockSpec((_K, _D), lambda i: (0, 0)),
        ],
        out_specs=pl.BlockSpec((_BN, 1), lambda i: (i, 0)),
        out_shape=jax.ShapeDtypeStruct((n, 1), jnp.int32),
    )(inputs, W)
    idx_rows = idx2d.reshape(n // _CH, _CH)

    w_round = W.astype(jnp.bfloat16).astype(jnp.float32)
    w_pad = jnp.pad(w_round, ((0, 0), (0, 2 * _D - _D)))
    z2 = inputs.reshape(n // 2, 2 * _D)
    q2, sqp = _make_sc_call()(w_pad, idx_rows, z2)
    q = q2.reshape(n, _D)

    loss, ppl = pl.pallas_call(
        _stats_kernel,
        grid=grid,
        in_specs=[
            pl.BlockSpec((_BN, 1), lambda i: (i, 0)),
            pl.BlockSpec((_NW, 16), lambda i: (0, 0)),
        ],
        out_specs=[
            pl.BlockSpec((1, 1), lambda i: (0, 0)),
            pl.BlockSpec((1, 1), lambda i: (0, 0)),
        ],
        out_shape=[
            jax.ShapeDtypeStruct((1, 1), jnp.float32),
            jax.ShapeDtypeStruct((1, 1), jnp.float32),
        ],
        scratch_shapes=[pltpu.VMEM((1, _K), jnp.float32)],
    )(idx2d, sqp)
    return q, loss[0, 0], ppl[0, 0]


# speedup vs baseline: 1.2641x; 1.2641x over previous
import functools

import jax
import jax.numpy as jnp
from jax import lax
from jax.experimental import pallas as pl
from jax.experimental.pallas import tpu as pltpu
from jax.experimental.pallas import tpu_sc as plsc

_K = 1024
_D = 64
_N = 16384
_COMMIT = 0.25
_BN = 1024

_NC = 2
_NS = 16
_NW = _NC * _NS
_BW = _N // _NW
_CH = 128
_NCH = _BW // _CH


def _argmin_kernel(z_ref, w_ref, idx_ref, loss_ref, ppl_ref,
                   counts_ref, sq_ref):
    i = pl.program_id(0)
    nblk = pl.num_programs(0)
    z = z_ref[...]
    w = w_ref[...]

    wm2 = w * (-2.0)
    s2 = jax.lax.dot_general(
        z, wm2, (((1,), (1,)), ((), ())), preferred_element_type=jnp.float32)
    zsq = jnp.sum(z * z, axis=1, keepdims=True)
    wsq = jnp.sum(w * w, axis=1)
    d = (zsq + wsq[None, :]) + s2

    dmin = jnp.min(d, axis=1, keepdims=True)
    col = jax.lax.broadcasted_iota(jnp.int32, d.shape, 1)
    idx = jnp.min(jnp.where(d == dmin, col, _K), axis=1)
    idx_ref[...] = idx.reshape(_BN // _CH, _CH)

    onehot = (col == idx[:, None]).astype(jnp.float32)
    ones_row = jnp.ones((1, _BN), jnp.float32)
    blk_counts = jax.lax.dot_general(
        ones_row, onehot, (((1,), (0,)), ((), ())),
        preferred_element_type=jnp.float32)
    blk_sq = jnp.sum(dmin)

    @pl.when(i == 0)
    def _init():
        counts_ref[...] = blk_counts
        sq_ref[0, 0] = blk_sq

    @pl.when(i > 0)
    def _acc():
        counts_ref[...] += blk_counts
        sq_ref[0, 0] += blk_sq

    @pl.when(i == nblk - 1)
    def _final():
        p = counts_ref[...] / jnp.float32(_N)
        ent = -jnp.sum(p * jnp.log(p + 1e-10))
        ppl_ref[...] = jnp.full((1, 1), jnp.exp(ent), jnp.float32)
        loss_ref[...] = jnp.full(
            (1, 1), (1.0 + _COMMIT) * sq_ref[0, 0] / jnp.float32(_N * _D),
            jnp.float32)


def _sc_gather(wp_hbm, idx_hbm, q2_hbm,
               idx_v, rbuf0, rbuf1, qbuf0, qbuf1, sem0, sem1, semw0, semw1):
    cid = lax.axis_index("c")
    sid = lax.axis_index("s")
    wid = sid * _NC + cid
    rbufs = [rbuf0, rbuf1]
    qbufs = [qbuf0, qbuf1]
    sems = [sem0, sem1]
    semws = [semw0, semw1]

    pltpu.sync_copy(idx_hbm.at[pl.ds(wid * _NCH, _NCH)], idx_v)

    copies = [None] * _NCH
    wcopies = [None] * _NCH
    copies[0] = pltpu.async_copy(wp_hbm.at[idx_v.at[0]], rbufs[0], sems[0])
    for j in range(_NCH):
        if j + 1 < _NCH:
            copies[j + 1] = pltpu.async_copy(
                wp_hbm.at[idx_v.at[j + 1]], rbufs[(j + 1) % 2],
                sems[(j + 1) % 2])
        copies[j].wait()
        if j >= 2:
            wcopies[j - 2].wait()
        rbuf = rbufs[j % 2]
        qbuf = qbufs[j % 2]

        def body(u, carry):
            for half in range(2):
                t = 2 * u + half
                for c in range(_D // 16):
                    qbuf[u, pl.ds(half * _D + c * 16, 16)] = (
                        rbuf[t, pl.ds(c * 16, 16)])
            return carry

        lax.fori_loop(0, _CH // 2, body, jnp.int32(0))
        wcopies[j] = pltpu.async_copy(
            qbuf, q2_hbm.at[pl.ds(wid * (_BW // 2) + j * (_CH // 2), _CH // 2)],
            semws[j % 2])
    wcopies[_NCH - 2].wait()
    wcopies[_NCH - 1].wait()


def _make_sc_call():
    mesh = plsc.VectorSubcoreMesh(core_axis_name="c", subcore_axis_name="s")
    return functools.partial(
        pl.kernel,
        mesh=mesh,
        out_type=jax.ShapeDtypeStruct((_N // 2, 2 * _D), jnp.float32),
        scratch_types=[
            pltpu.VMEM((_NCH, _CH), jnp.int32),
            pltpu.VMEM((_CH, 2 * _D), jnp.float32),
            pltpu.VMEM((_CH, 2 * _D), jnp.float32),
            pltpu.VMEM((_CH // 2, 2 * _D), jnp.float32),
            pltpu.VMEM((_CH // 2, 2 * _D), jnp.float32),
            pltpu.SemaphoreType.DMA,
            pltpu.SemaphoreType.DMA,
            pltpu.SemaphoreType.DMA,
            pltpu.SemaphoreType.DMA,
        ],
    )(_sc_gather)


def kernel(inputs, W):
    n = inputs.shape[0]
    grid = (n // _BN,)
    idx128, loss, ppl = pl.pallas_call(
        _argmin_kernel,
        grid=grid,
        in_specs=[
            pl.BlockSpec((_BN, _D), lambda i: (i, 0)),
            pl.BlockSpec((_K, _D), lambda i: (0, 0)),
        ],
        out_specs=[
            pl.BlockSpec((_BN // _CH, _CH), lambda i: (i, 0)),
            pl.BlockSpec((1, 1), lambda i: (0, 0)),
            pl.BlockSpec((1, 1), lambda i: (0, 0)),
        ],
        out_shape=[
            jax.ShapeDtypeStruct((n // _CH, _CH), jnp.int32),
            jax.ShapeDtypeStruct((1, 1), jnp.float32),
            jax.ShapeDtypeStruct((1, 1), jnp.float32),
        ],
        scratch_shapes=[
            pltpu.VMEM((1, _K), jnp.float32),
            pltpu.SMEM((1, 1), jnp.float32),
        ],
    )(inputs, W)

    w_round = W.astype(jnp.bfloat16).astype(jnp.float32)
    w_pad = jnp.pad(w_round, ((0, 0), (0, 2 * _D - _D)))
    q2 = _make_sc_call()(w_pad, idx128)
    q = q2.reshape(n, _D)
    return q, loss[0, 0], ppl[0, 0]


# speedup vs baseline: 1.9548x; 1.5463x over previous
import jax
import jax.numpy as jnp
from jax.experimental import pallas as pl
from jax.experimental.pallas import tpu as pltpu

_K = 1024
_D = 64
_COMMIT = 0.25
_BN = 2048


def _vq_kernel(z_ref, w_ref, q_ref, loss_ref, ppl_ref, counts_ref, sq_ref):
    i = pl.program_id(0)
    nblk = pl.num_programs(0)
    z = z_ref[...]
    w = w_ref[...]

    wm2 = w * (-2.0)
    s2 = jax.lax.dot_general(
        z, wm2, (((1,), (1,)), ((), ())), preferred_element_type=jnp.float32)
    zsq = jnp.sum(z * z, axis=1, keepdims=True)
    wsq = jnp.sum(w * w, axis=1)
    d = (zsq + wsq[None, :]) + s2

    idx = jnp.argmin(d, axis=1).astype(jnp.int32)

    col = jax.lax.broadcasted_iota(jnp.int32, d.shape, 1)
    onehot = (col == idx[:, None]).astype(jnp.float32)
    q = jax.lax.dot_general(
        onehot, w, (((1,), (0,)), ((), ())), preferred_element_type=jnp.float32)
    q_ref[...] = q

    ones_row = jnp.ones((1, _BN), jnp.float32)
    blk_counts = jax.lax.dot_general(
        ones_row, onehot, (((1,), (0,)), ((), ())),
        preferred_element_type=jnp.float32)
    diff = q - z
    blk_sq = jnp.sum(diff * diff)

    @pl.when(i == 0)
    def _init():
        counts_ref[...] = blk_counts
        sq_ref[0, 0] = blk_sq

    @pl.when(i > 0)
    def _acc():
        counts_ref[...] += blk_counts
        sq_ref[0, 0] += blk_sq

    @pl.when(i == nblk - 1)
    def _final():
        n_total = (nblk * _BN)
        mse = sq_ref[0, 0] / jnp.float32(n_total * _D)
        loss_ref[...] = jnp.full((1, 1), (1.0 + _COMMIT) * mse, jnp.float32)
        p = counts_ref[...] / jnp.float32(n_total)
        ent = -jnp.sum(p * jnp.log(p + 1e-10))
        ppl_ref[...] = jnp.full((1, 1), jnp.exp(ent), jnp.float32)


def kernel(inputs, W):
    n = inputs.shape[0]
    grid = (n // _BN,)
    q, loss, ppl = pl.pallas_call(
        _vq_kernel,
        grid=grid,
        in_specs=[
            pl.BlockSpec((_BN, _D), lambda i: (i, 0)),
            pl.BlockSpec((_K, _D), lambda i: (0, 0)),
        ],
        out_specs=[
            pl.BlockSpec((_BN, _D), lambda i: (i, 0)),
            pl.BlockSpec((1, 1), lambda i: (0, 0)),
            pl.BlockSpec((1, 1), lambda i: (0, 0)),
        ],
        out_shape=[
            jax.ShapeDtypeStruct((n, _D), jnp.float32),
            jax.ShapeDtypeStruct((1, 1), jnp.float32),
            jax.ShapeDtypeStruct((1, 1), jnp.float32),
        ],
        scratch_shapes=[
            pltpu.VMEM((1, _K), jnp.float32),
            pltpu.SMEM((1, 1), jnp.float32),
        ],
    )(inputs, W)
    return q, loss[0, 0], ppl[0, 0]
